# Initial kernel scaffold; baseline (speedup 1.0000x reference)
import jax
import jax.numpy as jnp
from jax.experimental import pallas as pl


def kernel(x, edge_index, W1_rel, b1, W1_root, W2_rel, b2, W2_root, W3_rel, b3, W3_root, W_lin, b_lin):
    raise NotImplementedError("write your pallas kernel here")


# speedup vs baseline: 4.0435x; 4.0435x over previous
import functools

import jax
import jax.numpy as jnp
from jax import lax
from jax.experimental import pallas as pl
from jax.experimental.pallas import tpu as pltpu
from jax.experimental.pallas import tpu_sc as plsc

NC = 2
NS = 16
K = 128


def _segment_sum_sc(srcr, dstr, t0, t1):
    _, _, C, _, _ = srcr.shape
    n_rows, d2 = t0.shape
    step = -(-(n_rows + 1) // (8 * NS)) * 8
    rows_pad = step * NS
    mesh = plsc.VectorSubcoreMesh(core_axis_name="c", subcore_axis_name="s")

    @functools.partial(
        pl.kernel,
        out_type=jax.ShapeDtypeStruct((NC, rows_pad, d2), jnp.float32),
        mesh=mesh,
        scratch_types=[
            pltpu.VMEM((1, K), jnp.int32),
            pltpu.VMEM((1, K), jnp.int32),
            pltpu.VMEM((K, d2), jnp.float32),
            pltpu.VMEM_SHARED((rows_pad, d2), jnp.float32),
            pltpu.SemaphoreType.DMA,
        ],
    )
    def seg_kernel(src_h, dst_h, t0_h, t1_h, z_h, out_h,
                   idx_s, idx_d, rows, agg, sem):
        c = lax.axis_index("c")
        s = lax.axis_index("s")
        pltpu.sync_copy(z_h, agg.at[pl.ds(s * step, step)])
        plsc.subcore_barrier()

        def body(j, carry):
            pltpu.sync_copy(src_h.at[c, s, j], idx_s)
            pltpu.sync_copy(dst_h.at[c, s, j], idx_d)

            @pl.when(c == 0)
            def _():
                pltpu.async_copy(t0_h.at[idx_s.at[0]], rows, sem).wait()

            @pl.when(c == 1)
            def _():
                pltpu.async_copy(t1_h.at[idx_s.at[0]], rows, sem).wait()

            pltpu.sync_copy(rows, agg.at[idx_d.at[0]], add=True)
            return carry

        lax.fori_loop(0, C, body, 0)
        plsc.subcore_barrier()
        pltpu.sync_copy(agg.at[pl.ds(s * step, step)],
                        out_h.at[c, pl.ds(s * step, step)])

    zeros = jnp.zeros((step, d2), jnp.float32)
    return seg_kernel(srcr, dstr, t0, t1, zeros)


def _chunk_edges(src, dst, n_rows, n_workers):
    n_edges = src.shape[0]
    epw = -(-n_edges // (n_workers * K)) * K
    e_pad = epw * n_workers
    c_chunks = epw // K
    src = jnp.concatenate(
        [src, jnp.zeros((e_pad - n_edges,), jnp.int32)])
    dst = jnp.concatenate(
        [dst, jnp.full((e_pad - n_edges,), n_rows, jnp.int32)])
    return (src.reshape(n_workers, c_chunks, 1, K),
            dst.reshape(n_workers, c_chunks, 1, K))


def _dotT(a, w):
    return lax.dot_general(a, w, (((1,), (1,)), ((), ())),
                           preferred_element_type=jnp.float32)


def _conv_first(agg, x, wrel, b, wroot, bn=1000):
    n_rows = x.shape[0]
    h_dim = wrel.shape[0]
    d2 = agg.shape[2]

    def body(agg_ref, x_ref, wrel_ref, b_ref, wroot_ref, out_ref):
        a = agg_ref[0] + agg_ref[1]
        h = _dotT(a, wrel_ref[...]) + _dotT(x_ref[...], wroot_ref[...])
        h = jnp.maximum(h + b_ref[...], 0.0)
        out_ref[0] = h[:, : h_dim // 2]
        out_ref[1] = h[:, h_dim // 2:]

    return pl.pallas_call(
        body,
        grid=(n_rows // bn,),
        in_specs=[
            pl.BlockSpec((2, bn, d2), lambda i: (0, i, 0)),
            pl.BlockSpec((bn, x.shape[1]), lambda i: (i, 0)),
            pl.BlockSpec(wrel.shape, lambda i: (0, 0)),
            pl.BlockSpec((1, h_dim), lambda i: (0, 0)),
            pl.BlockSpec(wroot.shape, lambda i: (0, 0)),
        ],
        out_specs=pl.BlockSpec((2, bn, h_dim // 2), lambda i: (0, i, 0)),
        out_shape=jax.ShapeDtypeStruct((2, n_rows, h_dim // 2), jnp.float32),
    )(agg, x, wrel, b.reshape(1, -1), wroot)


def _conv_mid(agg, hprev, wrel, b, wroot, bn=1000):
    n_rows = hprev.shape[1]
    h_dim = wrel.shape[0]
    d2 = agg.shape[2]

    def body(agg_ref, hp_ref, wrel_ref, b_ref, wroot_ref, out_ref):
        a = jnp.concatenate([agg_ref[0], agg_ref[1]], axis=-1)
        xp = jnp.concatenate([hp_ref[0], hp_ref[1]], axis=-1)
        h = _dotT(a, wrel_ref[...]) + _dotT(xp, wroot_ref[...])
        h = jnp.maximum(h + b_ref[...], 0.0)
        out_ref[0] = h[:, : h_dim // 2]
        out_ref[1] = h[:, h_dim // 2:]

    return pl.pallas_call(
        body,
        grid=(n_rows // bn,),
        in_specs=[
            pl.BlockSpec((2, bn, d2), lambda i: (0, i, 0)),
            pl.BlockSpec((2, bn, hprev.shape[2]), lambda i: (0, i, 0)),
            pl.BlockSpec(wrel.shape, lambda i: (0, 0)),
            pl.BlockSpec((1, h_dim), lambda i: (0, 0)),
            pl.BlockSpec(wroot.shape, lambda i: (0, 0)),
        ],
        out_specs=pl.BlockSpec((2, bn, h_dim // 2), lambda i: (0, i, 0)),
        out_shape=jax.ShapeDtypeStruct((2, n_rows, h_dim // 2), jnp.float32),
    )(agg, hprev, wrel, b.reshape(1, -1), wroot)


def _conv_last(agg, hprev, wrel, b, wroot, wlin, blin, bn=1000):
    n_rows = hprev.shape[1]
    h_dim = wrel.shape[0]
    d2 = agg.shape[2]
    o_dim = wlin.shape[0]

    def body(agg_ref, hp_ref, wrel_ref, b_ref, wroot_ref, wlin_ref, blin_ref,
             out_ref):
        a = jnp.concatenate([agg_ref[0], agg_ref[1]], axis=-1)
        xp = jnp.concatenate([hp_ref[0], hp_ref[1]], axis=-1)
        h = _dotT(a, wrel_ref[...]) + _dotT(xp, wroot_ref[...])
        h = jnp.maximum(h + b_ref[...], 0.0)
        out_ref[...] = _dotT(h, wlin_ref[...]) + blin_ref[...]

    return pl.pallas_call(
        body,
        grid=(n_rows // bn,),
        in_specs=[
            pl.BlockSpec((2, bn, d2), lambda i: (0, i, 0)),
            pl.BlockSpec((2, bn, hprev.shape[2]), lambda i: (0, i, 0)),
            pl.BlockSpec(wrel.shape, lambda i: (0, 0)),
            pl.BlockSpec((1, h_dim), lambda i: (0, 0)),
            pl.BlockSpec(wroot.shape, lambda i: (0, 0)),
            pl.BlockSpec(wlin.shape, lambda i: (0, 0)),
            pl.BlockSpec((1, o_dim), lambda i: (0, 0)),
        ],
        out_specs=pl.BlockSpec((bn, o_dim), lambda i: (i, 0)),
        out_shape=jax.ShapeDtypeStruct((n_rows, o_dim), jnp.float32),
    )(agg, hprev, wrel, b.reshape(1, -1), wroot, wlin, blin.reshape(1, -1))


def kernel(x, edge_index, W1_rel, b1, W1_root, W2_rel, b2, W2_root,
           W3_rel, b3, W3_root, W_lin, b_lin):
    n_rows, _ = x.shape
    src = edge_index[0]
    dst = edge_index[1]

    s1, d1 = _chunk_edges(src, dst, n_rows, NC * NS)
    srcr1 = s1.reshape(NC, NS, -1, 1, K)
    dstr1 = d1.reshape(NC, NS, -1, 1, K)
    s2, d2_ = _chunk_edges(src, dst, n_rows, NS)
    srcr2 = jnp.broadcast_to(s2, (NC,) + s2.shape)
    dstr2 = jnp.broadcast_to(d2_, (NC,) + d2_.shape)

    agg1 = _segment_sum_sc(srcr1, dstr1, x, x)
    h1 = _conv_first(agg1, x, W1_rel, b1, W1_root)

    agg2 = _segment_sum_sc(srcr2, dstr2, h1[0], h1[1])
    h2 = _conv_mid(agg2, h1, W2_rel, b2, W2_root)

    agg3 = _segment_sum_sc(srcr2, dstr2, h2[0], h2[1])
    return _conv_last(agg3, h2, W3_rel, b3, W3_root, W_lin, b_lin)


# speedup vs baseline: 6.2192x; 1.5381x over previous
import functools

import jax
import jax.numpy as jnp
from jax import lax
from jax.experimental import pallas as pl
from jax.experimental.pallas import tpu as pltpu
from jax.experimental.pallas import tpu_sc as plsc

NC = 2
NS = 16
K = 128


def _segment_sum_sc(edges, t0, t1):
    _, _, C, _, _ = edges.shape
    n_rows, d2 = t0.shape
    step = -(-(n_rows + 1) // (8 * NS)) * 8
    rows_pad = step * NS
    mesh = plsc.VectorSubcoreMesh(core_axis_name="c", subcore_axis_name="s")

    @functools.partial(
        pl.kernel,
        out_type=jax.ShapeDtypeStruct((NC, rows_pad, d2), jnp.float32),
        mesh=mesh,
        scratch_types=[
            pltpu.VMEM((3, 2, K), jnp.int32),
            pltpu.VMEM((2, K, d2), jnp.float32),
            pltpu.VMEM_SHARED((rows_pad, d2), jnp.float32),
            pltpu.SemaphoreType.DMA,
            pltpu.SemaphoreType.DMA,
            pltpu.SemaphoreType.DMA,
        ],
    )
    def seg_kernel(e_h, t0_h, t1_h, z_h, out_h,
                   idx, rows, agg, sem_i, sem_g, sem_s):
        c = lax.axis_index("c")
        s = lax.axis_index("s")

        def issue_idx(j, ip):
            pltpu.async_copy(e_h.at[c, s, j], idx.at[ip], sem_i)

        def wait_idx():
            pltpu.make_async_copy(e_h.at[c, s, 0], idx.at[0], sem_i).wait()

        def issue_gather(ip, p):
            @pl.when(c == 0)
            def _():
                pltpu.async_copy(t0_h.at[idx.at[ip, 0]], rows.at[p], sem_g)

            @pl.when(c == 1)
            def _():
                pltpu.async_copy(t1_h.at[idx.at[ip, 0]], rows.at[p], sem_g)

        def wait_gather():
            pltpu.make_async_copy(
                t0_h.at[idx.at[0, 0]], rows.at[0], sem_g).wait()

        def issue_scatter(ip, p):
            pltpu.async_copy(rows.at[p], agg.at[idx.at[ip, 1]], sem_s,
                             add=True)

        def wait_scatter():
            pltpu.make_async_copy(
                rows.at[0], agg.at[idx.at[0, 1]], sem_s).wait()

        issue_idx(0, 0)
        pltpu.sync_copy(z_h, agg.at[pl.ds(s * step, step)])
        wait_idx()
        issue_idx(1, 1)
        plsc.subcore_barrier()

        issue_gather(0, 0)
        wait_idx()
        issue_idx(2, 2)
        wait_gather()
        issue_gather(1, 1)
        issue_scatter(0, 0)

        def body(j, carry):
            wait_idx()
            wait_scatter()
            wait_gather()
            issue_gather(lax.rem(j + 1, 3), lax.rem(j + 1, 2))
            issue_scatter(lax.rem(j, 3), lax.rem(j, 2))
            issue_idx(j + 2, lax.rem(j + 2, 3))
            return carry

        lax.fori_loop(1, C - 2, body, 0)

        wait_idx()
        wait_scatter()
        wait_gather()
        issue_gather((C - 1) % 3, (C - 1) % 2)
        issue_scatter((C - 2) % 3, (C - 2) % 2)
        wait_scatter()
        wait_gather()
        issue_scatter((C - 1) % 3, (C - 1) % 2)
        wait_scatter()

        plsc.subcore_barrier()
        pltpu.sync_copy(agg.at[pl.ds(s * step, step)],
                        out_h.at[c, pl.ds(s * step, step)])

    zeros = jnp.zeros((step, d2), jnp.float32)
    return seg_kernel(edges, t0, t1, zeros)


def _chunk_edges(src, dst, n_rows, n_workers):
    n_edges = src.shape[0]
    epw = -(-n_edges // (n_workers * K)) * K
    e_pad = epw * n_workers
    c_chunks = epw // K
    src = jnp.concatenate(
        [src, jnp.zeros((e_pad - n_edges,), jnp.int32)])
    dst = jnp.concatenate(
        [dst, jnp.full((e_pad - n_edges,), n_rows, jnp.int32)])
    return jnp.stack(
        [src.reshape(n_workers, c_chunks, K),
         dst.reshape(n_workers, c_chunks, K)], axis=2)


def _dotT(a, w):
    return lax.dot_general(a, w, (((1,), (1,)), ((), ())),
                           preferred_element_type=jnp.float32)


def _conv_first(agg, x, wrel, b, wroot, bn=1000):
    n_rows = x.shape[0]
    h_dim = wrel.shape[0]
    d2 = agg.shape[2]

    def body(agg_ref, x_ref, wrel_ref, b_ref, wroot_ref, out_ref):
        a = agg_ref[0] + agg_ref[1]
        h = _dotT(a, wrel_ref[...]) + _dotT(x_ref[...], wroot_ref[...])
        h = jnp.maximum(h + b_ref[...], 0.0)
        out_ref[0] = h[:, : h_dim // 2]
        out_ref[1] = h[:, h_dim // 2:]

    return pl.pallas_call(
        body,
        grid=(n_rows // bn,),
        in_specs=[
            pl.BlockSpec((2, bn, d2), lambda i: (0, i, 0)),
            pl.BlockSpec((bn, x.shape[1]), lambda i: (i, 0)),
            pl.BlockSpec(wrel.shape, lambda i: (0, 0)),
            pl.BlockSpec((1, h_dim), lambda i: (0, 0)),
            pl.BlockSpec(wroot.shape, lambda i: (0, 0)),
        ],
        out_specs=pl.BlockSpec((2, bn, h_dim // 2), lambda i: (0, i, 0)),
        out_shape=jax.ShapeDtypeStruct((2, n_rows, h_dim // 2), jnp.float32),
    )(agg, x, wrel, b.reshape(1, -1), wroot)


def _conv_mid(agg, hprev, wrel, b, wroot, bn=1000):
    n_rows = hprev.shape[1]
    h_dim = wrel.shape[0]
    d2 = agg.shape[2]

    def body(agg_ref, hp_ref, wrel_ref, b_ref, wroot_ref, out_ref):
        a = jnp.concatenate([agg_ref[0], agg_ref[1]], axis=-1)
        xp = jnp.concatenate([hp_ref[0], hp_ref[1]], axis=-1)
        h = _dotT(a, wrel_ref[...]) + _dotT(xp, wroot_ref[...])
        h = jnp.maximum(h + b_ref[...], 0.0)
        out_ref[0] = h[:, : h_dim // 2]
        out_ref[1] = h[:, h_dim // 2:]

    return pl.pallas_call(
        body,
        grid=(n_rows // bn,),
        in_specs=[
            pl.BlockSpec((2, bn, d2), lambda i: (0, i, 0)),
            pl.BlockSpec((2, bn, hprev.shape[2]), lambda i: (0, i, 0)),
            pl.BlockSpec(wrel.shape, lambda i: (0, 0)),
            pl.BlockSpec((1, h_dim), lambda i: (0, 0)),
            pl.BlockSpec(wroot.shape, lambda i: (0, 0)),
        ],
        out_specs=pl.BlockSpec((2, bn, h_dim // 2), lambda i: (0, i, 0)),
        out_shape=jax.ShapeDtypeStruct((2, n_rows, h_dim // 2), jnp.float32),
    )(agg, hprev, wrel, b.reshape(1, -1), wroot)


def _conv_last(agg, hprev, wrel, b, wroot, wlin, blin, bn=1000):
    n_rows = hprev.shape[1]
    h_dim = wrel.shape[0]
    d2 = agg.shape[2]
    o_dim = wlin.shape[0]

    def body(agg_ref, hp_ref, wrel_ref, b_ref, wroot_ref, wlin_ref, blin_ref,
             out_ref):
        a = jnp.concatenate([agg_ref[0], agg_ref[1]], axis=-1)
        xp = jnp.concatenate([hp_ref[0], hp_ref[1]], axis=-1)
        h = _dotT(a, wrel_ref[...]) + _dotT(xp, wroot_ref[...])
        h = jnp.maximum(h + b_ref[...], 0.0)
        out_ref[...] = _dotT(h, wlin_ref[...]) + blin_ref[...]

    return pl.pallas_call(
        body,
        grid=(n_rows // bn,),
        in_specs=[
            pl.BlockSpec((2, bn, d2), lambda i: (0, i, 0)),
            pl.BlockSpec((2, bn, hprev.shape[2]), lambda i: (0, i, 0)),
            pl.BlockSpec(wrel.shape, lambda i: (0, 0)),
            pl.BlockSpec((1, h_dim), lambda i: (0, 0)),
            pl.BlockSpec(wroot.shape, lambda i: (0, 0)),
            pl.BlockSpec(wlin.shape, lambda i: (0, 0)),
            pl.BlockSpec((1, o_dim), lambda i: (0, 0)),
        ],
        out_specs=pl.BlockSpec((bn, o_dim), lambda i: (i, 0)),
        out_shape=jax.ShapeDtypeStruct((n_rows, o_dim), jnp.float32),
    )(agg, hprev, wrel, b.reshape(1, -1), wroot, wlin, blin.reshape(1, -1))


def kernel(x, edge_index, W1_rel, b1, W1_root, W2_rel, b2, W2_root,
           W3_rel, b3, W3_root, W_lin, b_lin):
    n_rows, _ = x.shape
    src = edge_index[0]
    dst = edge_index[1]

    e1 = _chunk_edges(src, dst, n_rows, NC * NS)
    e1 = e1.reshape(NC, NS, -1, 2, K)
    e2 = _chunk_edges(src, dst, n_rows, NS)
    e2 = jnp.broadcast_to(e2, (NC,) + e2.shape)

    agg1 = _segment_sum_sc(e1, x, x)
    h1 = _conv_first(agg1, x, W1_rel, b1, W1_root)

    agg2 = _segment_sum_sc(e2, h1[0], h1[1])
    h2 = _conv_mid(agg2, h1, W2_rel, b2, W2_root)

    agg3 = _segment_sum_sc(e2, h2[0], h2[1])
    return _conv_last(agg3, h2, W3_rel, b3, W3_root, W_lin, b_lin)


# speedup vs baseline: 7.2590x; 1.1672x over previous
import functools

import jax
import jax.numpy as jnp
from jax import lax
from jax.experimental import pallas as pl
from jax.experimental.pallas import tpu as pltpu
from jax.experimental.pallas import tpu_sc as plsc

NC = 2
NS = 16
K = 128


def _segment_sum_sc(edges, t0, t1):
    _, _, C, _, _ = edges.shape
    n_rows, d2 = t0.shape
    step = -(-(n_rows + 1) // (8 * NS)) * 8
    rows_pad = step * NS
    mesh = plsc.VectorSubcoreMesh(core_axis_name="c", subcore_axis_name="s")

    @functools.partial(
        pl.kernel,
        out_type=jax.ShapeDtypeStruct((NC, rows_pad, d2), jnp.float32),
        mesh=mesh,
        scratch_types=[
            pltpu.VMEM((4, 2, K), jnp.int32),
            pltpu.VMEM((3, K, d2), jnp.float32),
            pltpu.VMEM_SHARED((rows_pad, d2), jnp.float32),
            pltpu.SemaphoreType.DMA,
            pltpu.SemaphoreType.DMA((3,)),
            pltpu.SemaphoreType.DMA,
        ],
    )
    def seg_kernel(e_h, t0_h, t1_h, z_h, out_h,
                   idx, rows, agg, sem_i, sem_g, sem_s):
        c = lax.axis_index("c")
        s = lax.axis_index("s")

        def issue_idx(j, ip):
            pltpu.async_copy(e_h.at[c, s, j], idx.at[ip], sem_i)

        def wait_idx():
            pltpu.make_async_copy(e_h.at[c, s, 0], idx.at[0], sem_i).wait()

        def issue_gather(ip, p):
            @pl.when(c == 0)
            def _():
                pltpu.async_copy(t0_h.at[idx.at[ip, 0]], rows.at[p],
                                 sem_g.at[p])

            @pl.when(c == 1)
            def _():
                pltpu.async_copy(t1_h.at[idx.at[ip, 0]], rows.at[p],
                                 sem_g.at[p])

        def wait_gather(p):
            pltpu.make_async_copy(
                t0_h.at[idx.at[0, 0]], rows.at[p], sem_g.at[p]).wait()

        def issue_scatter(ip, p):
            pltpu.async_copy(rows.at[p], agg.at[idx.at[ip, 1]], sem_s,
                             add=True)

        def wait_scatter():
            pltpu.make_async_copy(
                rows.at[0], agg.at[idx.at[0, 1]], sem_s).wait()

        issue_idx(0, 0)
        pltpu.sync_copy(z_h, agg.at[pl.ds(s * step, step)])
        wait_idx()
        issue_idx(1, 1)
        plsc.subcore_barrier()

        issue_gather(0, 0)
        wait_idx()
        issue_gather(1, 1)
        issue_idx(2, 2)
        wait_idx()
        issue_gather(2, 2)
        issue_idx(3, 3)
        wait_gather(0)
        issue_scatter(0, 0)

        def body(j, carry):
            wait_gather(lax.rem(j, 3))
            wait_scatter()
            issue_scatter(lax.rem(j, 4), lax.rem(j, 3))
            wait_idx()
            issue_gather(lax.rem(j + 2, 4), lax.rem(j + 2, 3))
            issue_idx(j + 3, lax.rem(j + 3, 4))
            return carry

        lax.fori_loop(1, C - 3, body, 0)

        wait_gather((C - 3) % 3)
        wait_scatter()
        issue_scatter((C - 3) % 4, (C - 3) % 3)
        wait_idx()
        issue_gather((C - 1) % 4, (C - 1) % 3)
        wait_gather((C - 2) % 3)
        wait_scatter()
        issue_scatter((C - 2) % 4, (C - 2) % 3)
        wait_gather((C - 1) % 3)
        wait_scatter()
        issue_scatter((C - 1) % 4, (C - 1) % 3)
        wait_scatter()

        plsc.subcore_barrier()
        pltpu.sync_copy(agg.at[pl.ds(s * step, step)],
                        out_h.at[c, pl.ds(s * step, step)])

    zeros = jnp.zeros((step, d2), jnp.float32)
    return seg_kernel(edges, t0, t1, zeros)


def _chunk_edges(src, dst, n_rows, n_workers):
    n_edges = src.shape[0]
    epw = -(-n_edges // (n_workers * K)) * K
    e_pad = epw * n_workers
    c_chunks = epw // K
    src = jnp.concatenate(
        [src, jnp.zeros((e_pad - n_edges,), jnp.int32)])
    dst = jnp.concatenate(
        [dst, jnp.full((e_pad - n_edges,), n_rows, jnp.int32)])
    return jnp.stack(
        [src.reshape(n_workers, c_chunks, K),
         dst.reshape(n_workers, c_chunks, K)], axis=2)


def _dotT(a, w):
    return lax.dot_general(a, w, (((1,), (1,)), ((), ())),
                           preferred_element_type=jnp.float32)


def _conv_first(agg, x, wrel, b, wroot, bn=1000):
    n_rows = x.shape[0]
    h_dim = wrel.shape[0]
    d2 = agg.shape[2]

    def body(agg_ref, x_ref, wrel_ref, b_ref, wroot_ref, out_ref):
        a = agg_ref[0] + agg_ref[1]
        h = _dotT(a, wrel_ref[...]) + _dotT(x_ref[...], wroot_ref[...])
        h = jnp.maximum(h + b_ref[...], 0.0)
        out_ref[0] = h[:, : h_dim // 2]
        out_ref[1] = h[:, h_dim // 2:]

    return pl.pallas_call(
        body,
        grid=(n_rows // bn,),
        in_specs=[
            pl.BlockSpec((2, bn, d2), lambda i: (0, i, 0)),
            pl.BlockSpec((bn, x.shape[1]), lambda i: (i, 0)),
            pl.BlockSpec(wrel.shape, lambda i: (0, 0)),
            pl.BlockSpec((1, h_dim), lambda i: (0, 0)),
            pl.BlockSpec(wroot.shape, lambda i: (0, 0)),
        ],
        out_specs=pl.BlockSpec((2, bn, h_dim // 2), lambda i: (0, i, 0)),
        out_shape=jax.ShapeDtypeStruct((2, n_rows, h_dim // 2), jnp.float32),
    )(agg, x, wrel, b.reshape(1, -1), wroot)


def _conv_mid(agg, hprev, wrel, b, wroot, bn=1000):
    n_rows = hprev.shape[1]
    h_dim = wrel.shape[0]
    d2 = agg.shape[2]

    def body(agg_ref, hp_ref, wrel_ref, b_ref, wroot_ref, out_ref):
        a = jnp.concatenate([agg_ref[0], agg_ref[1]], axis=-1)
        xp = jnp.concatenate([hp_ref[0], hp_ref[1]], axis=-1)
        h = _dotT(a, wrel_ref[...]) + _dotT(xp, wroot_ref[...])
        h = jnp.maximum(h + b_ref[...], 0.0)
        out_ref[0] = h[:, : h_dim // 2]
        out_ref[1] = h[:, h_dim // 2:]

    return pl.pallas_call(
        body,
        grid=(n_rows // bn,),
        in_specs=[
            pl.BlockSpec((2, bn, d2), lambda i: (0, i, 0)),
            pl.BlockSpec((2, bn, hprev.shape[2]), lambda i: (0, i, 0)),
            pl.BlockSpec(wrel.shape, lambda i: (0, 0)),
            pl.BlockSpec((1, h_dim), lambda i: (0, 0)),
            pl.BlockSpec(wroot.shape, lambda i: (0, 0)),
        ],
        out_specs=pl.BlockSpec((2, bn, h_dim // 2), lambda i: (0, i, 0)),
        out_shape=jax.ShapeDtypeStruct((2, n_rows, h_dim // 2), jnp.float32),
    )(agg, hprev, wrel, b.reshape(1, -1), wroot)


def _conv_last(agg, hprev, wrel, b, wroot, wlin, blin, bn=1000):
    n_rows = hprev.shape[1]
    h_dim = wrel.shape[0]
    d2 = agg.shape[2]
    o_dim = wlin.shape[0]

    def body(agg_ref, hp_ref, wrel_ref, b_ref, wroot_ref, wlin_ref, blin_ref,
             out_ref):
        a = jnp.concatenate([agg_ref[0], agg_ref[1]], axis=-1)
        xp = jnp.concatenate([hp_ref[0], hp_ref[1]], axis=-1)
        h = _dotT(a, wrel_ref[...]) + _dotT(xp, wroot_ref[...])
        h = jnp.maximum(h + b_ref[...], 0.0)
        out_ref[...] = _dotT(h, wlin_ref[...]) + blin_ref[...]

    return pl.pallas_call(
        body,
        grid=(n_rows // bn,),
        in_specs=[
            pl.BlockSpec((2, bn, d2), lambda i: (0, i, 0)),
            pl.BlockSpec((2, bn, hprev.shape[2]), lambda i: (0, i, 0)),
            pl.BlockSpec(wrel.shape, lambda i: (0, 0)),
            pl.BlockSpec((1, h_dim), lambda i: (0, 0)),
            pl.BlockSpec(wroot.shape, lambda i: (0, 0)),
            pl.BlockSpec(wlin.shape, lambda i: (0, 0)),
            pl.BlockSpec((1, o_dim), lambda i: (0, 0)),
        ],
        out_specs=pl.BlockSpec((bn, o_dim), lambda i: (i, 0)),
        out_shape=jax.ShapeDtypeStruct((n_rows, o_dim), jnp.float32),
    )(agg, hprev, wrel, b.reshape(1, -1), wroot, wlin, blin.reshape(1, -1))


def kernel(x, edge_index, W1_rel, b1, W1_root, W2_rel, b2, W2_root,
           W3_rel, b3, W3_root, W_lin, b_lin):
    n_rows, _ = x.shape
    src = edge_index[0]
    dst = edge_index[1]

    e1 = _chunk_edges(src, dst, n_rows, NC * NS)
    e1 = e1.reshape(NC, NS, -1, 2, K)
    e2 = _chunk_edges(src, dst, n_rows, NS)
    e2 = jnp.broadcast_to(e2, (NC,) + e2.shape)

    agg1 = _segment_sum_sc(e1, x, x)
    h1 = _conv_first(agg1, x, W1_rel, b1, W1_root)

    agg2 = _segment_sum_sc(e2, h1[0], h1[1])
    h2 = _conv_mid(agg2, h1, W2_rel, b2, W2_root)

    agg3 = _segment_sum_sc(e2, h2[0], h2[1])
    return _conv_last(agg3, h2, W3_rel, b3, W3_root, W_lin, b_lin)


# speedup vs baseline: 12.1841x; 1.6785x over previous
import functools

import jax
import jax.numpy as jnp
from jax import lax
from jax.experimental import pallas as pl
from jax.experimental.pallas import tpu as pltpu
from jax.experimental.pallas import tpu_sc as plsc

NC = 2
NS = 16
K = 128


def _segment_sum_sc(edges, t0, t1):
    C = edges.shape[-3]
    per_core = edges.ndim == 5
    n_rows, d2 = t0.shape
    step = -(-(n_rows + 1) // (8 * NS)) * 8
    rows_pad = step * NS
    mesh = plsc.VectorSubcoreMesh(core_axis_name="c", subcore_axis_name="s")

    @functools.partial(
        pl.kernel,
        out_type=jax.ShapeDtypeStruct((NC, rows_pad, d2), jnp.float32),
        mesh=mesh,
        scratch_types=[
            pltpu.VMEM((4, 2, K), jnp.int32),
            pltpu.VMEM((3, K, d2), jnp.float32),
            pltpu.VMEM_SHARED((rows_pad, d2), jnp.float32),
            pltpu.SemaphoreType.DMA,
            pltpu.SemaphoreType.DMA((3,)),
            pltpu.SemaphoreType.DMA,
            pltpu.SemaphoreType.DMA,
        ],
    )
    def seg_kernel(e_h, t0_h, t1_h, z_h, out_h,
                   idx, rows, agg, sem_i, sem_g, sem_s, sem_z):
        c = lax.axis_index("c")
        s = lax.axis_index("s")

        def eslice(j):
            return e_h.at[c, s, j] if per_core else e_h.at[s, j]

        def issue_idx(j, ip):
            pltpu.async_copy(eslice(j), idx.at[ip], sem_i)

        def wait_idx():
            pltpu.make_async_copy(eslice(0), idx.at[0], sem_i).wait()

        def issue_gather(ip, p):
            @pl.when(c == 0)
            def _():
                pltpu.async_copy(t0_h.at[idx.at[ip, 0]], rows.at[p],
                                 sem_g.at[p])

            @pl.when(c == 1)
            def _():
                pltpu.async_copy(t1_h.at[idx.at[ip, 0]], rows.at[p],
                                 sem_g.at[p])

        def wait_gather(p):
            pltpu.make_async_copy(
                t0_h.at[idx.at[0, 0]], rows.at[p], sem_g.at[p]).wait()

        def issue_scatter(ip, p):
            pltpu.async_copy(rows.at[p], agg.at[idx.at[ip, 1]], sem_s,
                             add=True)

        def wait_scatter():
            pltpu.make_async_copy(
                rows.at[0], agg.at[idx.at[0, 1]], sem_s).wait()

        issue_idx(0, 0)
        zero_dma = pltpu.async_copy(
            z_h, agg.at[pl.ds(s * step, step)], sem_z)
        wait_idx()
        issue_idx(1, 1)
        issue_gather(0, 0)
        wait_idx()
        issue_idx(2, 2)
        issue_gather(1, 1)
        wait_idx()
        issue_idx(3, 3)
        issue_gather(2, 2)
        zero_dma.wait()
        plsc.subcore_barrier()
        wait_gather(0)
        issue_scatter(0, 0)

        def body(j, carry):
            wait_gather(lax.rem(j, 3))
            wait_scatter()
            issue_scatter(lax.rem(j, 4), lax.rem(j, 3))
            wait_idx()
            issue_gather(lax.rem(j + 2, 4), lax.rem(j + 2, 3))
            issue_idx(j + 3, lax.rem(j + 3, 4))
            return carry

        lax.fori_loop(1, C - 3, body, 0)

        wait_gather((C - 3) % 3)
        wait_scatter()
        issue_scatter((C - 3) % 4, (C - 3) % 3)
        wait_idx()
        issue_gather((C - 1) % 4, (C - 1) % 3)
        wait_gather((C - 2) % 3)
        wait_scatter()
        issue_scatter((C - 2) % 4, (C - 2) % 3)
        wait_gather((C - 1) % 3)
        wait_scatter()
        issue_scatter((C - 1) % 4, (C - 1) % 3)
        wait_scatter()

        plsc.subcore_barrier()
        pltpu.sync_copy(agg.at[pl.ds(s * step, step)],
                        out_h.at[c, pl.ds(s * step, step)])

    zeros = jnp.zeros((step, d2), jnp.float32)
    return seg_kernel(edges, t0, t1, zeros)


def _chunk_edges(src, dst, n_rows, n_workers):
    n_edges = src.shape[0]
    epw = -(-n_edges // (n_workers * K)) * K
    e_pad = epw * n_workers
    c_chunks = epw // K
    pad = e_pad - n_edges
    cyc = jnp.arange(pad, dtype=jnp.int32) % 96
    src = jnp.concatenate([src, cyc])
    dst = jnp.concatenate([dst, n_rows + cyc])
    return jnp.stack(
        [src.reshape(n_workers, c_chunks, K),
         dst.reshape(n_workers, c_chunks, K)], axis=2)


def _dotT(a, w):
    return lax.dot_general(a, w, (((1,), (1,)), ((), ())),
                           preferred_element_type=jnp.float32)


def _conv_first(agg, x, wrel, b, wroot, bn=1000):
    n_rows = x.shape[0]
    h_dim = wrel.shape[0]
    d2 = agg.shape[2]

    def body(agg_ref, x_ref, wrel_ref, b_ref, wroot_ref, o0_ref, o1_ref):
        a = agg_ref[0] + agg_ref[1]
        h = _dotT(a, wrel_ref[...]) + _dotT(x_ref[...], wroot_ref[...])
        h = jnp.maximum(h + b_ref[...], 0.0)
        o0_ref[...] = h[:, : h_dim // 2]
        o1_ref[...] = h[:, h_dim // 2:]

    return pl.pallas_call(
        body,
        grid=(n_rows // bn,),
        in_specs=[
            pl.BlockSpec((2, bn, d2), lambda i: (0, i, 0)),
            pl.BlockSpec((bn, x.shape[1]), lambda i: (i, 0)),
            pl.BlockSpec(wrel.shape, lambda i: (0, 0)),
            pl.BlockSpec((1, h_dim), lambda i: (0, 0)),
            pl.BlockSpec(wroot.shape, lambda i: (0, 0)),
        ],
        out_specs=[pl.BlockSpec((bn, h_dim // 2), lambda i: (i, 0)),
                   pl.BlockSpec((bn, h_dim // 2), lambda i: (i, 0))],
        out_shape=[jax.ShapeDtypeStruct((n_rows, h_dim // 2), jnp.float32),
                   jax.ShapeDtypeStruct((n_rows, h_dim // 2), jnp.float32)],
    )(agg, x, wrel, b.reshape(1, -1), wroot)


def _conv_mid(agg, hp0, hp1, wrel, b, wroot, bn=1000):
    n_rows = hp0.shape[0]
    h_dim = wrel.shape[0]
    d2 = agg.shape[2]

    def body(agg_ref, hp0_ref, hp1_ref, wrel_ref, b_ref, wroot_ref,
             o0_ref, o1_ref):
        a = jnp.concatenate([agg_ref[0], agg_ref[1]], axis=-1)
        xp = jnp.concatenate([hp0_ref[...], hp1_ref[...]], axis=-1)
        h = _dotT(a, wrel_ref[...]) + _dotT(xp, wroot_ref[...])
        h = jnp.maximum(h + b_ref[...], 0.0)
        o0_ref[...] = h[:, : h_dim // 2]
        o1_ref[...] = h[:, h_dim // 2:]

    return pl.pallas_call(
        body,
        grid=(n_rows // bn,),
        in_specs=[
            pl.BlockSpec((2, bn, d2), lambda i: (0, i, 0)),
            pl.BlockSpec((bn, hp0.shape[1]), lambda i: (i, 0)),
            pl.BlockSpec((bn, hp1.shape[1]), lambda i: (i, 0)),
            pl.BlockSpec(wrel.shape, lambda i: (0, 0)),
            pl.BlockSpec((1, h_dim), lambda i: (0, 0)),
            pl.BlockSpec(wroot.shape, lambda i: (0, 0)),
        ],
        out_specs=[pl.BlockSpec((bn, h_dim // 2), lambda i: (i, 0)),
                   pl.BlockSpec((bn, h_dim // 2), lambda i: (i, 0))],
        out_shape=[jax.ShapeDtypeStruct((n_rows, h_dim // 2), jnp.float32),
                   jax.ShapeDtypeStruct((n_rows, h_dim // 2), jnp.float32)],
    )(agg, hp0, hp1, wrel, b.reshape(1, -1), wroot)


def _conv_last(agg, hp0, hp1, wrel, b, wroot, wlin, blin, bn=1000):
    n_rows = hp0.shape[0]
    h_dim = wrel.shape[0]
    d2 = agg.shape[2]
    o_dim = wlin.shape[0]

    def body(agg_ref, hp0_ref, hp1_ref, wrel_ref, b_ref, wroot_ref,
             wlin_ref, blin_ref, out_ref):
        a = jnp.concatenate([agg_ref[0], agg_ref[1]], axis=-1)
        xp = jnp.concatenate([hp0_ref[...], hp1_ref[...]], axis=-1)
        h = _dotT(a, wrel_ref[...]) + _dotT(xp, wroot_ref[...])
        h = jnp.maximum(h + b_ref[...], 0.0)
        out_ref[...] = _dotT(h, wlin_ref[...]) + blin_ref[...]

    return pl.pallas_call(
        body,
        grid=(n_rows // bn,),
        in_specs=[
            pl.BlockSpec((2, bn, d2), lambda i: (0, i, 0)),
            pl.BlockSpec((bn, hp0.shape[1]), lambda i: (i, 0)),
            pl.BlockSpec((bn, hp1.shape[1]), lambda i: (i, 0)),
            pl.BlockSpec(wrel.shape, lambda i: (0, 0)),
            pl.BlockSpec((1, h_dim), lambda i: (0, 0)),
            pl.BlockSpec(wroot.shape, lambda i: (0, 0)),
            pl.BlockSpec(wlin.shape, lambda i: (0, 0)),
            pl.BlockSpec((1, o_dim), lambda i: (0, 0)),
        ],
        out_specs=pl.BlockSpec((bn, o_dim), lambda i: (i, 0)),
        out_shape=jax.ShapeDtypeStruct((n_rows, o_dim), jnp.float32),
    )(agg, hp0, hp1, wrel, b.reshape(1, -1), wroot, wlin,
      blin.reshape(1, -1))


def kernel(x, edge_index, W1_rel, b1, W1_root, W2_rel, b2, W2_root,
           W3_rel, b3, W3_root, W_lin, b_lin):
    n_rows, _ = x.shape
    src = edge_index[0]
    dst = edge_index[1]

    e1 = _chunk_edges(src, dst, n_rows, NC * NS)
    e1 = e1.reshape(NC, NS, -1, 2, K)
    e2 = _chunk_edges(src, dst, n_rows, NS)

    agg1 = _segment_sum_sc(e1, x, x)
    h1a, h1b = _conv_first(agg1, x, W1_rel, b1, W1_root)

    agg2 = _segment_sum_sc(e2, h1a, h1b)
    h2a, h2b = _conv_mid(agg2, h1a, h1b, W2_rel, b2, W2_root)

    agg3 = _segment_sum_sc(e2, h2a, h2b)
    return _conv_last(agg3, h2a, h2b, W3_rel, b3, W3_root, W_lin, b_lin)
